# baseline (speedup 1.0000x reference)
import functools

import jax
import jax.numpy as jnp
from jax import lax
from jax.experimental import pallas as pl
from jax.experimental.pallas import tpu as pltpu
from jax.experimental.pallas import tpu_sc as plsc

N = 4096
D = 128
K = 16
SH = 9
RB = 42
RCUT = 32.0
WIN = 8
NSPAT = 9
BLK = 256
CPAD = 128


def _masked_dist(coord_ref, cxt_ref, cyt_ref, czt_ref, r0):
    cx = coord_ref[:, 0:1]
    cy = coord_ref[:, 1:2]
    cz = coord_ref[:, 2:3]
    dx = cx - cxt_ref[...]
    dy = cy - cyt_ref[...]
    dz = cz - czt_ref[...]
    d = dx * dx + dy * dy + dz * dz
    gi = r0 + lax.broadcasted_iota(jnp.int32, (BLK, 1), 0)
    j = lax.broadcasted_iota(jnp.int32, (BLK, N), 1)
    d = jnp.where(jnp.abs(gi - j) <= WIN, jnp.float32(jnp.inf), d)
    return d, gi, j


def _argmin_tiebreak_low(d, j):
    m = jnp.min(d, axis=1, keepdims=True)
    return jnp.min(jnp.where(d <= m, j, N), axis=1, keepdims=True)


def _knn_interior_body(coord_ref, cxt_ref, cyt_ref, czt_ref, nei_ref):
    r0 = (pl.program_id(0) + 1) * BLK
    d, gi, j = _masked_dist(coord_ref, cxt_ref, cyt_ref, czt_ref, r0)
    spat0 = _argmin_tiebreak_low(d, j)
    t = lax.broadcasted_iota(jnp.int32, (BLK, K), 1)
    idx_seq = gi - WIN + 1 + t + (t >= WIN - 1).astype(jnp.int32)
    nei_ref[...] = jnp.where(t <= K - 2, idx_seq, spat0)


def _knn_edge_body(coord_ref, cxt_ref, cyt_ref, czt_ref, nei_ref):
    r0 = pl.program_id(0) * (N - BLK)
    d, gi, j = _masked_dist(coord_ref, cxt_ref, cyt_ref, czt_ref, r0)
    spat = []
    for _ in range(NSPAT):
        am = _argmin_tiebreak_low(d, j)
        spat.append(am)
        d = jnp.where(j == am, jnp.float32(jnp.inf), d)
    t = lax.broadcasted_iota(jnp.int32, (BLK, K), 1)
    l = jnp.minimum(gi, WIN)
    r = jnp.minimum(N - 1 - gi, WIN)
    s = l + r
    idx_seq = gi - l + 1 + t + (t >= l - 1).astype(jnp.int32)
    q = t - (s - 1)
    idx_spat = jnp.zeros((BLK, K), jnp.int32)
    for c in range(NSPAT):
        idx_spat = jnp.where(q == c, spat[c], idx_spat)
    nei_ref[...] = jnp.where(t <= s - 2, idx_seq, idx_spat)


def _knn_indices(coord):
    coordT = coord.T
    row_specs = lambda imap: [
        pl.BlockSpec((BLK, 3), imap),
        pl.BlockSpec((1, N), lambda i: (0, 0)),
        pl.BlockSpec((1, N), lambda i: (0, 0)),
        pl.BlockSpec((1, N), lambda i: (0, 0)),
    ]
    args = (coord, coordT[0:1], coordT[1:2], coordT[2:3])
    interior = pl.pallas_call(
        _knn_interior_body,
        grid=(N // BLK - 2,),
        in_specs=row_specs(lambda i: (i + 1, 0)),
        out_specs=pl.BlockSpec((BLK, K), lambda i: (i, 0)),
        out_shape=jax.ShapeDtypeStruct((N - 2 * BLK, K), jnp.int32),
    )(*args)
    edge = pl.pallas_call(
        _knn_edge_body,
        grid=(2,),
        in_specs=row_specs(lambda i: (i * (N // BLK - 1), 0)),
        out_specs=pl.BlockSpec((BLK, K), lambda i: (i, 0)),
        out_shape=jax.ShapeDtypeStruct((2 * BLK, K), jnp.int32),
    )(*args)
    return jnp.concatenate([edge[:BLK], interior, edge[BLK:]], axis=0)


def _make_sc_gather():
    info = plsc.get_sparse_core_info()
    nc, ns = info.num_cores, info.num_subcores
    nw = nc * ns
    b_per_w = (N * K) // nw
    chunk = 128
    nchunks = b_per_w // chunk
    mesh = plsc.VectorSubcoreMesh(core_axis_name="c", subcore_axis_name="s")

    @functools.partial(
        pl.kernel,
        mesh=mesh,
        out_type=(
            jax.ShapeDtypeStruct((N * K, D), jnp.float32),
            jax.ShapeDtypeStruct((N * K, CPAD), jnp.float32),
        ),
        scratch_types=[
            pltpu.VMEM((chunk,), jnp.int32),
            pltpu.VMEM((chunk, D), jnp.float32),
            pltpu.VMEM((chunk, CPAD), jnp.float32),
            pltpu.SemaphoreType.DMA,
            pltpu.SemaphoreType.DMA,
        ],
    )
    def gather(feat_hbm, cpad_hbm, idx_hbm, gf_hbm, gc_hbm,
               idx_v, rows_v, crows_v, sem_f, sem_c):
        wid = lax.axis_index("s") * nc + lax.axis_index("c")
        base = wid * b_per_w
        for ch in range(nchunks):
            off = base + ch * chunk
            pltpu.sync_copy(idx_hbm.at[pl.ds(off, chunk)], idx_v)
            cp_f = pltpu.async_copy(feat_hbm.at[idx_v], rows_v, sem_f)
            cp_c = pltpu.async_copy(cpad_hbm.at[idx_v], crows_v, sem_c)
            cp_f.wait()
            cp_c.wait()
            pltpu.sync_copy(rows_v, gf_hbm.at[pl.ds(off, chunk)])
            pltpu.sync_copy(crows_v, gc_hbm.at[pl.ds(off, chunk)])

    return gather


_sc_gather = None


def _gather_neighbors(features, cpad, nei_flat):
    global _sc_gather
    if _sc_gather is None:
        _sc_gather = _make_sc_gather()
    return _sc_gather(features, cpad, nei_flat)


def _msg_body(coord_ref, cpad_ref, feat_ref, nf_ref, nc_ref,
              wtp_ref, wang_ref, bmsg_ref, w1m_ref, w1r_ref, w1f_ref,
              b1_ref, w2_ref, b2_ref, wmove_ref, gamma_ref,
              out_ref, ncoord_ref):
    NK = BLK * K
    f = feat_ref[...]
    nf = nf_ref[...]
    ncrd = nc_ref[...]
    own = cpad_ref[...]
    vec = ncrd - jnp.broadcast_to(
        own[:, None, :], (BLK, K, CPAD)).reshape(NK, CPAD)

    x = vec[:, 0:1]
    y = vec[:, 1:2]
    z = vec[:, 2:3]
    nsq = x * x + y * y + z * z
    rr = jnp.sqrt(jnp.where(nsq == 0.0, 1.0, nsq))

    s3 = jnp.sqrt(jnp.float32(3.0))
    s15 = jnp.sqrt(jnp.float32(15.0))
    s5 = jnp.sqrt(jnp.float32(5.0))
    ang = [
        jnp.ones_like(x),
        s3 * x, s3 * y, s3 * z,
        s15 * x * y, s15 * y * z,
        (s5 / 2.0) * (2.0 * z * z - x * x - y * y),
        s15 * x * z, (s15 / 2.0) * (x * x - y * y),
    ]

    acc = jnp.zeros((NK, D), jnp.float32)
    for s in range(SH):
        acc = acc + jnp.dot(ang[s] * nf, wtp_ref[s],
                            preferred_element_type=jnp.float32)
    ang9 = jnp.concatenate(ang, axis=1)
    messages = acc + jnp.dot(ang9, wang_ref[...],
                             preferred_element_type=jnp.float32)
    messages = messages + bmsg_ref[...]

    t = rr * (1.0 / RCUT)
    irow = (1 + lax.broadcasted_iota(jnp.int32, (1, RB), 1)).astype(jnp.float32)
    u = irow * jnp.minimum(t, 1.0)
    v = u - 2.0 * jnp.round(u * 0.5)
    w = v * v
    p = jnp.float32(-0.00614086361689008)
    for c in (0.08086620765133497, -0.5986450252875573, 2.5500285767157873,
              -5.167702006048083, 3.1415925160351934):
        p = p * w + jnp.float32(c)
    rad = jnp.where(t < 1.0, v * p, 0.0)

    ff = jnp.dot(f, w1f_ref[...], preferred_element_type=jnp.float32)
    ff = jnp.broadcast_to(ff[:, None, :], (BLK, K, RB)).reshape(NK, RB)
    h = (jnp.dot(messages, w1m_ref[...], preferred_element_type=jnp.float32)
         + jnp.dot(rad, w1r_ref[...], preferred_element_type=jnp.float32)
         + ff + b1_ref[...])
    h = h * jax.nn.sigmoid(h)
    mix = jnp.dot(h, w2_ref[...], preferred_element_type=jnp.float32)
    mix = mix + b2_ref[...]

    fn = (messages * mix).reshape(BLK, K, D).sum(axis=1) * (1.0 / K)

    xo = f + fn
    mu = jnp.mean(xo, axis=-1, keepdims=True)
    xc = xo - mu
    var = jnp.mean(xc * xc, axis=-1, keepdims=True)
    out_ref[...] = gamma_ref[...] * xc * lax.rsqrt(var + 1e-5)
    ncoord_ref[...] = coord_ref[...] + 0.001 * jnp.dot(
        fn, wmove_ref[...], preferred_element_type=jnp.float32)


def _msg_pass(coord, cpad, features, gf, gc,
              W_tp, W_ang_p, b_msg, W1m, W1r, W1f, b1, W2, b2, W_move, gamma):
    nb = N // BLK
    full = lambda *shape: pl.BlockSpec(shape, lambda i: (0,) * len(shape))
    return pl.pallas_call(
        _msg_body,
        grid=(nb,),
        in_specs=[
            pl.BlockSpec((BLK, 3), lambda i: (i, 0)),
            pl.BlockSpec((BLK, CPAD), lambda i: (i, 0)),
            pl.BlockSpec((BLK, D), lambda i: (i, 0)),
            pl.BlockSpec((BLK * K, D), lambda i: (i, 0)),
            pl.BlockSpec((BLK * K, CPAD), lambda i: (i, 0)),
            full(SH, D, D),
            full(SH, D),
            full(1, D),
            full(D, RB),
            full(RB, RB),
            full(D, RB),
            full(1, RB),
            full(RB, D),
            full(1, D),
            full(D, 3),
            full(1, D),
        ],
        out_specs=[
            pl.BlockSpec((BLK, D), lambda i: (i, 0)),
            pl.BlockSpec((BLK, 3), lambda i: (i, 0)),
        ],
        out_shape=[
            jax.ShapeDtypeStruct((N, D), jnp.float32),
            jax.ShapeDtypeStruct((N, 3), jnp.float32),
        ],
    )(coord, cpad, features, gf, gc,
      W_tp, W_ang_p, b_msg.reshape(1, D), W1m, W1r, W1f, b1.reshape(1, RB),
      W2, b2.reshape(1, D), W_move, gamma.reshape(1, D))


def kernel(coord, mask, features, W_tp, W_ang, b_msg, W1, b1, W2, b2,
           W_move, gamma):
    del mask
    nei = _knn_indices(coord)
    cpad = jnp.pad(coord, ((0, 0), (0, CPAD - 3)))
    gf, gc = _gather_neighbors(features, cpad, nei.reshape(N * K))
    W1m, W1r, W1f = W1[:D], W1[D:D + RB], W1[D + RB:]
    out, new_coord = _msg_pass(
        coord, cpad, features, gf, gc,
        W_tp, W_ang, b_msg, W1m, W1r, W1f, b1, W2, b2, W_move, gamma)
    return out, new_coord


# speedup vs baseline: 1.0337x; 1.0337x over previous
import functools

import jax
import jax.numpy as jnp
from jax import lax
from jax.experimental import pallas as pl
from jax.experimental.pallas import tpu as pltpu
from jax.experimental.pallas import tpu_sc as plsc

N = 4096
D = 128
K = 16
SH = 9
RB = 42
RCUT = 32.0
WIN = 8
NSPAT = 9
BLK = 256
CPAD = 128


def _masked_dist(coord_ref, cxt_ref, cyt_ref, czt_ref, r0):
    cx = coord_ref[:, 0:1]
    cy = coord_ref[:, 1:2]
    cz = coord_ref[:, 2:3]
    dx = cx - cxt_ref[...]
    dy = cy - cyt_ref[...]
    dz = cz - czt_ref[...]
    d = dx * dx + dy * dy + dz * dz
    gi = r0 + lax.broadcasted_iota(jnp.int32, (BLK, 1), 0)
    j = lax.broadcasted_iota(jnp.int32, (BLK, N), 1)
    d = jnp.where(jnp.abs(gi - j) <= WIN, jnp.float32(jnp.inf), d)
    return d, gi, j


def _argmin_tiebreak_low(d, j):
    m = jnp.min(d, axis=1, keepdims=True)
    return jnp.min(jnp.where(d <= m, j, N), axis=1, keepdims=True)


def _knn_interior_body(coord_ref, cxt_ref, cyt_ref, czt_ref, nei_ref):
    r0 = (pl.program_id(0) + 1) * BLK
    d, gi, j = _masked_dist(coord_ref, cxt_ref, cyt_ref, czt_ref, r0)
    spat0 = _argmin_tiebreak_low(d, j)
    t = lax.broadcasted_iota(jnp.int32, (BLK, K), 1)
    idx_seq = gi - WIN + 1 + t + (t >= WIN - 1).astype(jnp.int32)
    nei_ref[...] = jnp.where(t <= K - 2, idx_seq, spat0)


def _knn_edge_body(coord_ref, cxt_ref, cyt_ref, czt_ref, nei_ref):
    r0 = pl.program_id(0) * (N - BLK)
    d, gi, j = _masked_dist(coord_ref, cxt_ref, cyt_ref, czt_ref, r0)
    spat = []
    for _ in range(NSPAT):
        am = _argmin_tiebreak_low(d, j)
        spat.append(am)
        d = jnp.where(j == am, jnp.float32(jnp.inf), d)
    t = lax.broadcasted_iota(jnp.int32, (BLK, K), 1)
    l = jnp.minimum(gi, WIN)
    r = jnp.minimum(N - 1 - gi, WIN)
    s = l + r
    idx_seq = gi - l + 1 + t + (t >= l - 1).astype(jnp.int32)
    q = t - (s - 1)
    idx_spat = jnp.zeros((BLK, K), jnp.int32)
    for c in range(NSPAT):
        idx_spat = jnp.where(q == c, spat[c], idx_spat)
    nei_ref[...] = jnp.where(t <= s - 2, idx_seq, idx_spat)


def _knn_indices(coord):
    coordT = coord.T
    row_specs = lambda imap: [
        pl.BlockSpec((BLK, 3), imap),
        pl.BlockSpec((1, N), lambda i: (0, 0)),
        pl.BlockSpec((1, N), lambda i: (0, 0)),
        pl.BlockSpec((1, N), lambda i: (0, 0)),
    ]
    args = (coord, coordT[0:1], coordT[1:2], coordT[2:3])
    interior = pl.pallas_call(
        _knn_interior_body,
        grid=(N // BLK - 2,),
        in_specs=row_specs(lambda i: (i + 1, 0)),
        out_specs=pl.BlockSpec((BLK, K), lambda i: (i, 0)),
        out_shape=jax.ShapeDtypeStruct((N - 2 * BLK, K), jnp.int32),
    )(*args)
    edge = pl.pallas_call(
        _knn_edge_body,
        grid=(2,),
        in_specs=row_specs(lambda i: (i * (N // BLK - 1), 0)),
        out_specs=pl.BlockSpec((BLK, K), lambda i: (i, 0)),
        out_shape=jax.ShapeDtypeStruct((2 * BLK, K), jnp.int32),
    )(*args)
    return jnp.concatenate([edge[:BLK], interior, edge[BLK:]], axis=0)


NG = N + 2 * BLK * K


def _make_sc_gather():
    info = plsc.get_sparse_core_info()
    nc, ns = info.num_cores, info.num_subcores
    nw = nc * ns
    b_per_w = NG // nw
    chunk = 128
    nchunks = b_per_w // chunk
    mesh = plsc.VectorSubcoreMesh(core_axis_name="c", subcore_axis_name="s")

    @functools.partial(
        pl.kernel,
        mesh=mesh,
        out_type=(
            jax.ShapeDtypeStruct((NG, D), jnp.float32),
            jax.ShapeDtypeStruct((NG, CPAD), jnp.float32),
        ),
        scratch_types=[
            pltpu.VMEM((chunk,), jnp.int32),
            pltpu.VMEM((chunk, D), jnp.float32),
            pltpu.VMEM((chunk, CPAD), jnp.float32),
            pltpu.SemaphoreType.DMA,
            pltpu.SemaphoreType.DMA,
        ],
    )
    def gather(feat_hbm, cpad_hbm, idx_hbm, gf_hbm, gc_hbm,
               idx_v, rows_v, crows_v, sem_f, sem_c):
        wid = lax.axis_index("s") * nc + lax.axis_index("c")
        base = wid * b_per_w
        for ch in range(nchunks):
            off = base + ch * chunk
            pltpu.sync_copy(idx_hbm.at[pl.ds(off, chunk)], idx_v)
            cp_f = pltpu.async_copy(feat_hbm.at[idx_v], rows_v, sem_f)
            cp_c = pltpu.async_copy(cpad_hbm.at[idx_v], crows_v, sem_c)
            cp_f.wait()
            cp_c.wait()
            pltpu.sync_copy(rows_v, gf_hbm.at[pl.ds(off, chunk)])
            pltpu.sync_copy(crows_v, gc_hbm.at[pl.ds(off, chunk)])

    return gather


_sc_gather = None


def _gather_neighbors(features, cpad, nei_flat):
    global _sc_gather
    if _sc_gather is None:
        _sc_gather = _make_sc_gather()
    return _sc_gather(features, cpad, nei_flat)


CW = 16


def _msg_tail(f, nf, vec, cown,
              wtp_ref, wang_ref, bmsg_ref, w1m_ref, w1r_ref, w1f_ref,
              b1_ref, w2_ref, b2_ref, wmove_ref, gamma_ref,
              out_ref, ncoord_ref):
    NK = BLK * K
    x = vec[:, 0:1]
    y = vec[:, 1:2]
    z = vec[:, 2:3]
    nsq = x * x + y * y + z * z
    rr = jnp.sqrt(jnp.where(nsq == 0.0, 1.0, nsq))

    s3 = jnp.sqrt(jnp.float32(3.0))
    s15 = jnp.sqrt(jnp.float32(15.0))
    s5 = jnp.sqrt(jnp.float32(5.0))
    ang = [
        jnp.ones_like(x),
        s3 * x, s3 * y, s3 * z,
        s15 * x * y, s15 * y * z,
        (s5 / 2.0) * (2.0 * z * z - x * x - y * y),
        s15 * x * z, (s15 / 2.0) * (x * x - y * y),
    ]

    acc = jnp.zeros((NK, D), jnp.float32)
    for s in range(SH):
        acc = acc + jnp.dot(ang[s] * nf, wtp_ref[s],
                            preferred_element_type=jnp.float32)
    ang9 = jnp.concatenate(ang, axis=1)
    messages = acc + jnp.dot(ang9, wang_ref[...],
                             preferred_element_type=jnp.float32)
    messages = messages + bmsg_ref[...]

    t = rr * (1.0 / RCUT)
    irow = (1 + lax.broadcasted_iota(jnp.int32, (1, RB), 1)).astype(jnp.float32)
    u = irow * jnp.minimum(t, 1.0)
    v = u - 2.0 * jnp.round(u * 0.5)
    w = v * v
    p = jnp.float32(-0.00614086361689008)
    for c in (0.08086620765133497, -0.5986450252875573, 2.5500285767157873,
              -5.167702006048083, 3.1415925160351934):
        p = p * w + jnp.float32(c)
    rad = jnp.where(t < 1.0, v * p, 0.0)

    ff = jnp.dot(f, w1f_ref[...], preferred_element_type=jnp.float32)
    ff = jnp.broadcast_to(ff[:, None, :], (BLK, K, RB)).reshape(NK, RB)
    h = (jnp.dot(messages, w1m_ref[...], preferred_element_type=jnp.float32)
         + jnp.dot(rad, w1r_ref[...], preferred_element_type=jnp.float32)
         + ff + b1_ref[...])
    h = h * jax.nn.sigmoid(h)
    mix = jnp.dot(h, w2_ref[...], preferred_element_type=jnp.float32)
    mix = mix + b2_ref[...]

    fn = (messages * mix).reshape(BLK, K, D).sum(axis=1) * (1.0 / K)

    xo = f + fn
    mu = jnp.mean(xo, axis=-1, keepdims=True)
    xc = xo - mu
    var = jnp.mean(xc * xc, axis=-1, keepdims=True)
    out_ref[...] = gamma_ref[...] * xc * lax.rsqrt(var + 1e-5)
    ncoord_ref[...] = cown + 0.001 * jnp.dot(
        fn, wmove_ref[...], preferred_element_type=jnp.float32)


_OFFS = [t - 7 if t <= 6 else t - 6 for t in range(K - 1)]


def _msg_interior_body(coord_ref, cpad_ref, feat_ref, spatf_ref, spatc_ref,
                       *wrefs):
    r0 = (pl.program_id(0) + 1) * BLK
    f = feat_ref[pl.ds(r0, BLK), :]
    own = cpad_ref[pl.ds(r0, BLK), :]
    nf_parts = [feat_ref[pl.ds(r0 + d, BLK), :][:, None, :] for d in _OFFS]
    nf_parts.append(spatf_ref[...][:, None, :])
    vc_parts = [cpad_ref[pl.ds(r0 + d, BLK), :][:, None, :] for d in _OFFS]
    vc_parts.append(spatc_ref[...][:, 0:CW][:, None, :])
    nf = jnp.concatenate(nf_parts, axis=1).reshape(BLK * K, D)
    ncrd = jnp.concatenate(vc_parts, axis=1).reshape(BLK * K, CW)
    vec = ncrd - jnp.broadcast_to(
        own[:, None, :], (BLK, K, CW)).reshape(BLK * K, CW)
    _msg_tail(f, nf, vec, coord_ref[...], *wrefs)


def _msg_edge_body(coord_ref, cpad_ref, feat_ref, gf_ref, gc_ref, *wrefs):
    r0 = pl.program_id(0) * (N - BLK)
    f = feat_ref[pl.ds(r0, BLK), :]
    own = cpad_ref[pl.ds(r0, BLK), :]
    nf = gf_ref[...]
    ncrd = gc_ref[...][:, 0:CW]
    vec = ncrd - jnp.broadcast_to(
        own[:, None, :], (BLK, K, CW)).reshape(BLK * K, CW)
    _msg_tail(f, nf, vec, coord_ref[...], *wrefs)


def _msg_pass(coord, cpad, features, gf, gc,
              W_tp, W_ang_p, b_msg, W1m, W1r, W1f, b1, W2, b2, W_move, gamma):
    full = lambda *shape: pl.BlockSpec(shape, lambda i: (0,) * len(shape))
    w_specs = [
        full(SH, D, D),
        full(SH, D),
        full(1, D),
        full(D, RB),
        full(RB, RB),
        full(D, RB),
        full(1, RB),
        full(RB, D),
        full(1, D),
        full(D, 3),
        full(1, D),
    ]
    w_args = (W_tp, W_ang_p, b_msg.reshape(1, D), W1m, W1r, W1f,
              b1.reshape(1, RB), W2, b2.reshape(1, D), W_move,
              gamma.reshape(1, D))
    out_specs = [
        pl.BlockSpec((BLK, D), lambda i: (i, 0)),
        pl.BlockSpec((BLK, 3), lambda i: (i, 0)),
    ]
    interior = pl.pallas_call(
        _msg_interior_body,
        grid=(N // BLK - 2,),
        in_specs=[
            pl.BlockSpec((BLK, 3), lambda i: (i + 1, 0)),
            full(N, CW),
            full(N, D),
            pl.BlockSpec((BLK, D), lambda i: (i + 1, 0)),
            pl.BlockSpec((BLK, CPAD), lambda i: (i + 1, 0)),
        ] + w_specs,
        out_specs=out_specs,
        out_shape=[
            jax.ShapeDtypeStruct((N - 2 * BLK, D), jnp.float32),
            jax.ShapeDtypeStruct((N - 2 * BLK, 3), jnp.float32),
        ],
    )(coord, cpad, features, gf, gc, *w_args)
    edge = pl.pallas_call(
        _msg_edge_body,
        grid=(2,),
        in_specs=[
            pl.BlockSpec((BLK, 3), lambda i: (i * (N // BLK - 1), 0)),
            full(N, CW),
            full(N, D),
            pl.BlockSpec((BLK * K, D), lambda i: (i + 1, 0)),
            pl.BlockSpec((BLK * K, CPAD), lambda i: (i + 1, 0)),
        ] + w_specs,
        out_specs=out_specs,
        out_shape=[
            jax.ShapeDtypeStruct((2 * BLK, D), jnp.float32),
            jax.ShapeDtypeStruct((2 * BLK, 3), jnp.float32),
        ],
    )(coord, cpad, features, gf, gc, *w_args)
    out = jnp.concatenate([edge[0][:BLK], interior[0], edge[0][BLK:]], axis=0)
    ncrd = jnp.concatenate([edge[1][:BLK], interior[1], edge[1][BLK:]], axis=0)
    return out, ncrd


def kernel(coord, mask, features, W_tp, W_ang, b_msg, W1, b1, W2, b2,
           W_move, gamma):
    del mask
    nei = _knn_indices(coord)
    cpad = jnp.pad(coord, ((0, 0), (0, CW - 3)))
    cpad_tab = jnp.pad(coord, ((0, 0), (0, CPAD - 3)))
    idx_all = jnp.concatenate([
        nei[:, K - 1],
        nei[:BLK].reshape(BLK * K),
        nei[N - BLK:].reshape(BLK * K),
    ])
    gf, gc = _gather_neighbors(features, cpad_tab, idx_all)
    W1m, W1r, W1f = W1[:D], W1[D:D + RB], W1[D + RB:]
    out, new_coord = _msg_pass(
        coord, cpad, features, gf, gc,
        W_tp, W_ang, b_msg, W1m, W1r, W1f, b1, W2, b2, W_move, gamma)
    return out, new_coord


# speedup vs baseline: 1.0493x; 1.0151x over previous
import functools

import jax
import jax.numpy as jnp
from jax import lax
from jax.experimental import pallas as pl
from jax.experimental.pallas import tpu as pltpu
from jax.experimental.pallas import tpu_sc as plsc

N = 4096
D = 128
K = 16
SH = 9
RB = 42
RCUT = 32.0
WIN = 8
NSPAT = 9
BLK = 256
CPAD = 128


def _masked_dist(coord_ref, cxt_ref, cyt_ref, czt_ref, r0):
    cx = coord_ref[:, 0:1]
    cy = coord_ref[:, 1:2]
    cz = coord_ref[:, 2:3]
    dx = cx - cxt_ref[...]
    dy = cy - cyt_ref[...]
    dz = cz - czt_ref[...]
    d = dx * dx + dy * dy + dz * dz
    gi = r0 + lax.broadcasted_iota(jnp.int32, (BLK, 1), 0)
    j = lax.broadcasted_iota(jnp.int32, (BLK, N), 1)
    d = jnp.where(jnp.abs(gi - j) <= WIN, jnp.float32(jnp.inf), d)
    return d, gi, j


def _argmin_tiebreak_low(d, j):
    m = jnp.min(d, axis=1, keepdims=True)
    return jnp.min(jnp.where(d <= m, j, N), axis=1, keepdims=True)


def _knn_body(coord_ref, cxt_ref, cyt_ref, czt_ref, nei_ref, d_ref, spat_ref):
    i = pl.program_id(0)
    r0 = i * BLK
    d, gi, j = _masked_dist(coord_ref, cxt_ref, cyt_ref, czt_ref, r0)
    spat0 = _argmin_tiebreak_low(d, j)
    spat_ref[:, 0:1] = spat0

    @pl.when(jnp.logical_or(i == 0, i == N // BLK - 1))
    def _extra_spatial():
        d_ref[...] = jnp.where(j == spat0, jnp.float32(jnp.inf), d)
        for c in range(1, NSPAT):
            am = _argmin_tiebreak_low(d_ref[...], j)
            spat_ref[:, c:c + 1] = am
            d_ref[...] = jnp.where(j == am, jnp.float32(jnp.inf), d_ref[...])

    t = lax.broadcasted_iota(jnp.int32, (BLK, K), 1)
    l = jnp.minimum(gi, WIN)
    r = jnp.minimum(N - 1 - gi, WIN)
    s = l + r
    idx_seq = gi - l + 1 + t + (t >= l - 1).astype(jnp.int32)
    q = t - (s - 1)
    sp = spat_ref[...]
    idx_spat = jnp.zeros((BLK, K), jnp.int32)
    for c in range(NSPAT):
        idx_spat = jnp.where(q == c, sp[:, c:c + 1], idx_spat)
    nei_ref[...] = jnp.where(t <= s - 2, idx_seq, idx_spat)


def _knn_indices(coord):
    coordT = coord.T
    return pl.pallas_call(
        _knn_body,
        grid=(N // BLK,),
        in_specs=[
            pl.BlockSpec((BLK, 3), lambda i: (i, 0)),
            pl.BlockSpec((1, N), lambda i: (0, 0)),
            pl.BlockSpec((1, N), lambda i: (0, 0)),
            pl.BlockSpec((1, N), lambda i: (0, 0)),
        ],
        out_specs=pl.BlockSpec((BLK, K), lambda i: (i, 0)),
        out_shape=jax.ShapeDtypeStruct((N, K), jnp.int32),
        scratch_shapes=[
            pltpu.VMEM((BLK, N), jnp.float32),
            pltpu.VMEM((BLK, K), jnp.int32),
        ],
    )(coord, coordT[0:1], coordT[1:2], coordT[2:3])


NG = N + 2 * BLK * K


def _make_sc_gather():
    info = plsc.get_sparse_core_info()
    nc, ns = info.num_cores, info.num_subcores
    nw = nc * ns
    b_per_w = NG // nw
    chunk = 128
    nchunks = b_per_w // chunk
    mesh = plsc.VectorSubcoreMesh(core_axis_name="c", subcore_axis_name="s")

    @functools.partial(
        pl.kernel,
        mesh=mesh,
        out_type=(
            jax.ShapeDtypeStruct((NG, D), jnp.float32),
            jax.ShapeDtypeStruct((NG, CPAD), jnp.float32),
        ),
        scratch_types=[
            pltpu.VMEM((chunk,), jnp.int32),
            pltpu.VMEM((chunk, D), jnp.float32),
            pltpu.VMEM((chunk, CPAD), jnp.float32),
            pltpu.SemaphoreType.DMA,
            pltpu.SemaphoreType.DMA,
        ],
    )
    def gather(feat_hbm, cpad_hbm, idx_hbm, gf_hbm, gc_hbm,
               idx_v, rows_v, crows_v, sem_f, sem_c):
        wid = lax.axis_index("s") * nc + lax.axis_index("c")
        base = wid * b_per_w
        for ch in range(nchunks):
            off = base + ch * chunk
            pltpu.sync_copy(idx_hbm.at[pl.ds(off, chunk)], idx_v)
            cp_f = pltpu.async_copy(feat_hbm.at[idx_v], rows_v, sem_f)
            cp_c = pltpu.async_copy(cpad_hbm.at[idx_v], crows_v, sem_c)
            cp_f.wait()
            cp_c.wait()
            pltpu.sync_copy(rows_v, gf_hbm.at[pl.ds(off, chunk)])
            pltpu.sync_copy(crows_v, gc_hbm.at[pl.ds(off, chunk)])

    return gather


_sc_gather = None


def _gather_neighbors(features, cpad, nei_flat):
    global _sc_gather
    if _sc_gather is None:
        _sc_gather = _make_sc_gather()
    return _sc_gather(features, cpad, nei_flat)


CW = 16


def _msg_tail(f, nf, vec, cown,
              wtp_ref, wang_ref, bmsg_ref, w1m_ref, w1r_ref, w1f_ref,
              b1_ref, w2_ref, b2_ref, wmove_ref, gamma_ref,
              out_ref, ncoord_ref):
    NK = BLK * K
    x = vec[:, 0:1]
    y = vec[:, 1:2]
    z = vec[:, 2:3]
    nsq = x * x + y * y + z * z
    rr = jnp.sqrt(jnp.where(nsq == 0.0, 1.0, nsq))

    s3 = jnp.sqrt(jnp.float32(3.0))
    s15 = jnp.sqrt(jnp.float32(15.0))
    s5 = jnp.sqrt(jnp.float32(5.0))
    ang = [
        jnp.ones_like(x),
        s3 * x, s3 * y, s3 * z,
        s15 * x * y, s15 * y * z,
        (s5 / 2.0) * (2.0 * z * z - x * x - y * y),
        s15 * x * z, (s15 / 2.0) * (x * x - y * y),
    ]

    acc = jnp.zeros((NK, D), jnp.float32)
    for s in range(SH):
        acc = acc + jnp.dot(ang[s] * nf, wtp_ref[s],
                            preferred_element_type=jnp.float32)
    ang9 = jnp.concatenate(ang, axis=1)
    messages = acc + jnp.dot(ang9, wang_ref[...],
                             preferred_element_type=jnp.float32)
    messages = messages + bmsg_ref[...]

    t = rr * (1.0 / RCUT)
    irow = (1 + lax.broadcasted_iota(jnp.int32, (1, RB), 1)).astype(jnp.float32)
    u = irow * jnp.minimum(t, 1.0)
    v = u - 2.0 * jnp.round(u * 0.5)
    w = v * v
    p = jnp.float32(-0.00614086361689008)
    for c in (0.08086620765133497, -0.5986450252875573, 2.5500285767157873,
              -5.167702006048083, 3.1415925160351934):
        p = p * w + jnp.float32(c)
    rad = jnp.where(t < 1.0, v * p, 0.0)

    ff = jnp.dot(f, w1f_ref[...], preferred_element_type=jnp.float32)
    ff = jnp.broadcast_to(ff[:, None, :], (BLK, K, RB)).reshape(NK, RB)
    h = (jnp.dot(messages, w1m_ref[...], preferred_element_type=jnp.float32)
         + jnp.dot(rad, w1r_ref[...], preferred_element_type=jnp.float32)
         + ff + b1_ref[...])
    h = h * jax.nn.sigmoid(h)
    mix = jnp.dot(h, w2_ref[...], preferred_element_type=jnp.float32)
    mix = mix + b2_ref[...]

    fn = (messages * mix).reshape(BLK, K, D).sum(axis=1) * (1.0 / K)

    xo = f + fn
    mu = jnp.mean(xo, axis=-1, keepdims=True)
    xc = xo - mu
    var = jnp.mean(xc * xc, axis=-1, keepdims=True)
    out_ref[...] = gamma_ref[...] * xc * lax.rsqrt(var + 1e-5)
    ncoord_ref[...] = cown + 0.001 * jnp.dot(
        fn, wmove_ref[...], preferred_element_type=jnp.float32)


_OFFS = [t - 7 if t <= 6 else t - 6 for t in range(K - 1)]


def _msg_body(coord_ref, cpad_ref, feat_ref, spatf_ref, spatc_ref,
              gfe_ref, gce_ref, wtp_ref, wang_ref, bmsg_ref, w1m_ref,
              w1r_ref, w1f_ref, b1_ref, w2_ref, b2_ref, wmove_ref, gamma_ref,
              out_ref, ncoord_ref, nf_s, vc_s):
    i = pl.program_id(0)
    r0 = i * BLK
    is_edge = jnp.logical_or(i == 0, i == N // BLK - 1)

    @pl.when(jnp.logical_not(is_edge))
    def _interior():
        for tslot, doff in enumerate(_OFFS):
            nf_s[:, tslot, :] = feat_ref[pl.ds(r0 + doff, BLK), :]
            vc_s[:, tslot, :] = cpad_ref[pl.ds(r0 + doff, BLK), :]
        nf_s[:, K - 1, :] = spatf_ref[...]
        vc_s[:, K - 1, :] = spatc_ref[...][:, 0:CW]

    @pl.when(is_edge)
    def _edge():
        nf_s[...] = gfe_ref[...].reshape(BLK, K, D)
        vc_s[...] = gce_ref[...][:, 0:CW].reshape(BLK, K, CW)

    f = feat_ref[pl.ds(r0, BLK), :]
    own = cpad_ref[pl.ds(r0, BLK), :]
    nf = nf_s[...].reshape(BLK * K, D)
    vec = (vc_s[...] - own[:, None, :]).reshape(BLK * K, CW)
    _msg_tail(f, nf, vec, coord_ref[...],
              wtp_ref, wang_ref, bmsg_ref, w1m_ref, w1r_ref, w1f_ref,
              b1_ref, w2_ref, b2_ref, wmove_ref, gamma_ref,
              out_ref, ncoord_ref)


def _msg_pass(coord, cpad, features, gf, gc,
              W_tp, W_ang_p, b_msg, W1m, W1r, W1f, b1, W2, b2, W_move, gamma):
    full = lambda *shape: pl.BlockSpec(shape, lambda i: (0,) * len(shape))
    nb = N // BLK
    edge_map = lambda i: (jnp.where(i == nb - 1, 2, 1), 0)
    w_specs = [
        full(SH, D, D),
        full(SH, D),
        full(1, D),
        full(D, RB),
        full(RB, RB),
        full(D, RB),
        full(1, RB),
        full(RB, D),
        full(1, D),
        full(D, 3),
        full(1, D),
    ]
    w_args = (W_tp, W_ang_p, b_msg.reshape(1, D), W1m, W1r, W1f,
              b1.reshape(1, RB), W2, b2.reshape(1, D), W_move,
              gamma.reshape(1, D))
    return pl.pallas_call(
        _msg_body,
        grid=(nb,),
        in_specs=[
            pl.BlockSpec((BLK, 3), lambda i: (i, 0)),
            full(N, CW),
            full(N, D),
            pl.BlockSpec((BLK, D), lambda i: (i, 0)),
            pl.BlockSpec((BLK, CPAD), lambda i: (i, 0)),
            pl.BlockSpec((BLK * K, D), edge_map),
            pl.BlockSpec((BLK * K, CPAD), edge_map),
        ] + w_specs,
        out_specs=[
            pl.BlockSpec((BLK, D), lambda i: (i, 0)),
            pl.BlockSpec((BLK, 3), lambda i: (i, 0)),
        ],
        out_shape=[
            jax.ShapeDtypeStruct((N, D), jnp.float32),
            jax.ShapeDtypeStruct((N, 3), jnp.float32),
        ],
        scratch_shapes=[
            pltpu.VMEM((BLK, K, D), jnp.float32),
            pltpu.VMEM((BLK, K, CW), jnp.float32),
        ],
    )(coord, cpad, features, gf, gc, gf, gc, *w_args)


def kernel(coord, mask, features, W_tp, W_ang, b_msg, W1, b1, W2, b2,
           W_move, gamma):
    del mask
    nei = _knn_indices(coord)
    cpad = jnp.pad(coord, ((0, 0), (0, CW - 3)))
    cpad_tab = jnp.pad(coord, ((0, 0), (0, CPAD - 3)))
    idx_all = jnp.concatenate([
        nei[:, K - 1],
        nei[:BLK].reshape(BLK * K),
        nei[N - BLK:].reshape(BLK * K),
    ])
    gf, gc = _gather_neighbors(features, cpad_tab, idx_all)
    W1m, W1r, W1f = W1[:D], W1[D:D + RB], W1[D + RB:]
    out, new_coord = _msg_pass(
        coord, cpad, features, gf, gc,
        W_tp, W_ang, b_msg, W1m, W1r, W1f, b1, W2, b2, W_move, gamma)
    return out, new_coord


# speedup vs baseline: 1.1347x; 1.0813x over previous
import functools

import jax
import jax.numpy as jnp
from jax import lax
from jax.experimental import pallas as pl
from jax.experimental.pallas import tpu as pltpu
from jax.experimental.pallas import tpu_sc as plsc

N = 4096
D = 128
K = 16
SH = 9
RB = 42
RCUT = 32.0
WIN = 8
NSPAT = 9
BLK = 256
CPAD = 128


def _masked_dist(coord_ref, cxt_ref, cyt_ref, czt_ref, r0):
    cx = coord_ref[:, 0:1]
    cy = coord_ref[:, 1:2]
    cz = coord_ref[:, 2:3]
    dx = cx - cxt_ref[...]
    dy = cy - cyt_ref[...]
    dz = cz - czt_ref[...]
    d = dx * dx + dy * dy + dz * dz
    gi = r0 + lax.broadcasted_iota(jnp.int32, (BLK, 1), 0)
    j = lax.broadcasted_iota(jnp.int32, (BLK, N), 1)
    d = jnp.where(jnp.abs(gi - j) <= WIN, jnp.float32(jnp.inf), d)
    return d, gi, j


def _argmin_tiebreak_low(d, j):
    m = jnp.min(d, axis=1, keepdims=True)
    return jnp.min(jnp.where(d <= m, j, N), axis=1, keepdims=True)


def _knn_body(coord_ref, cxt_ref, cyt_ref, czt_ref, nei_ref, d_ref, spat_ref):
    i = pl.program_id(0)
    r0 = i * BLK
    d, gi, j = _masked_dist(coord_ref, cxt_ref, cyt_ref, czt_ref, r0)
    spat0 = _argmin_tiebreak_low(d, j)
    spat_ref[:, 0:1] = spat0

    @pl.when(jnp.logical_or(i == 0, i == N // BLK - 1))
    def _extra_spatial():
        d_ref[...] = jnp.where(j == spat0, jnp.float32(jnp.inf), d)
        for c in range(1, NSPAT):
            am = _argmin_tiebreak_low(d_ref[...], j)
            spat_ref[:, c:c + 1] = am
            d_ref[...] = jnp.where(j == am, jnp.float32(jnp.inf), d_ref[...])

    t = lax.broadcasted_iota(jnp.int32, (BLK, K), 1)
    l = jnp.minimum(gi, WIN)
    r = jnp.minimum(N - 1 - gi, WIN)
    s = l + r
    idx_seq = gi - l + 1 + t + (t >= l - 1).astype(jnp.int32)
    q = t - (s - 1)
    sp = spat_ref[...]
    idx_spat = jnp.zeros((BLK, K), jnp.int32)
    for c in range(NSPAT):
        idx_spat = jnp.where(q == c, sp[:, c:c + 1], idx_spat)
    nei_ref[...] = jnp.where(t <= s - 2, idx_seq, idx_spat)


def _knn_indices(coord):
    coordT = coord.T
    return pl.pallas_call(
        _knn_body,
        grid=(N // BLK,),
        in_specs=[
            pl.BlockSpec((BLK, 3), lambda i: (i, 0)),
            pl.BlockSpec((1, N), lambda i: (0, 0)),
            pl.BlockSpec((1, N), lambda i: (0, 0)),
            pl.BlockSpec((1, N), lambda i: (0, 0)),
        ],
        out_specs=pl.BlockSpec((BLK, K), lambda i: (i, 0)),
        out_shape=jax.ShapeDtypeStruct((N, K), jnp.int32),
        scratch_shapes=[
            pltpu.VMEM((BLK, N), jnp.float32),
            pltpu.VMEM((BLK, K), jnp.int32),
        ],
    )(coord, coordT[0:1], coordT[1:2], coordT[2:3])


NG = N + 2 * BLK * K


def _make_sc_gather():
    info = plsc.get_sparse_core_info()
    nc, ns = info.num_cores, info.num_subcores
    nw = nc * ns
    b_per_w = NG // nw
    chunk = 128
    nchunks = b_per_w // chunk
    mesh = plsc.VectorSubcoreMesh(core_axis_name="c", subcore_axis_name="s")

    @functools.partial(
        pl.kernel,
        mesh=mesh,
        out_type=(
            jax.ShapeDtypeStruct((NG, D), jnp.float32),
            jax.ShapeDtypeStruct((NG, CPAD), jnp.float32),
        ),
        scratch_types=[
            pltpu.VMEM((chunk,), jnp.int32),
            pltpu.VMEM((chunk, D), jnp.float32),
            pltpu.VMEM((chunk, CPAD), jnp.float32),
            pltpu.SemaphoreType.DMA,
            pltpu.SemaphoreType.DMA,
        ],
    )
    def gather(feat_hbm, cpad_hbm, idx_hbm, gf_hbm, gc_hbm,
               idx_v, rows_v, crows_v, sem_f, sem_c):
        wid = lax.axis_index("s") * nc + lax.axis_index("c")
        base = wid * b_per_w
        for ch in range(nchunks):
            off = base + ch * chunk
            pltpu.sync_copy(idx_hbm.at[pl.ds(off, chunk)], idx_v)
            cp_f = pltpu.async_copy(feat_hbm.at[idx_v], rows_v, sem_f)
            cp_c = pltpu.async_copy(cpad_hbm.at[idx_v], crows_v, sem_c)
            cp_f.wait()
            cp_c.wait()
            pltpu.sync_copy(rows_v, gf_hbm.at[pl.ds(off, chunk)])
            pltpu.sync_copy(crows_v, gc_hbm.at[pl.ds(off, chunk)])

    return gather


_sc_gather = None


def _gather_neighbors(features, cpad, nei_flat):
    global _sc_gather
    if _sc_gather is None:
        _sc_gather = _make_sc_gather()
    return _sc_gather(features, cpad, nei_flat)


CW = 16


def _msg_tail(f, nf, vec, cown,
              wtp_ref, wang_ref, bmsg_ref, w1m_ref, w1r_ref, w1f_ref,
              b1_ref, w2_ref, b2_ref, wmove_ref, gamma_ref,
              out_ref, ncoord_ref):
    NK = BLK * K
    x = vec[:, 0:1]
    y = vec[:, 1:2]
    z = vec[:, 2:3]
    nsq = x * x + y * y + z * z
    rr = jnp.sqrt(jnp.where(nsq == 0.0, 1.0, nsq))

    s3 = jnp.sqrt(jnp.float32(3.0))
    s15 = jnp.sqrt(jnp.float32(15.0))
    s5 = jnp.sqrt(jnp.float32(5.0))
    ang = [
        jnp.ones_like(x),
        s3 * x, s3 * y, s3 * z,
        s15 * x * y, s15 * y * z,
        (s5 / 2.0) * (2.0 * z * z - x * x - y * y),
        s15 * x * z, (s15 / 2.0) * (x * x - y * y),
    ]

    nf16 = nf.astype(jnp.bfloat16)
    acc = jnp.zeros((NK, D), jnp.float32)
    for s in range(SH):
        acc = acc + jnp.dot(ang[s].astype(jnp.bfloat16) * nf16, wtp_ref[s],
                            preferred_element_type=jnp.float32)
    ang9 = jnp.concatenate(ang, axis=1)
    messages = acc + jnp.dot(ang9, wang_ref[...],
                             preferred_element_type=jnp.float32)
    messages = messages + bmsg_ref[...]

    t = rr * (1.0 / RCUT)
    irow = (1 + lax.broadcasted_iota(jnp.int32, (1, RB), 1)).astype(jnp.float32)
    u = irow * jnp.minimum(t, 1.0)
    v = u - 2.0 * jnp.round(u * 0.5)
    w = v * v
    p = jnp.float32(-0.00614086361689008)
    for c in (0.08086620765133497, -0.5986450252875573, 2.5500285767157873,
              -5.167702006048083, 3.1415925160351934):
        p = p * w + jnp.float32(c)
    rad = jnp.where(t < 1.0, v * p, 0.0)

    ff = jnp.dot(f, w1f_ref[...], preferred_element_type=jnp.float32)
    ff = jnp.broadcast_to(ff[:, None, :], (BLK, K, RB)).reshape(NK, RB)
    h = (jnp.dot(messages, w1m_ref[...], preferred_element_type=jnp.float32)
         + jnp.dot(rad, w1r_ref[...], preferred_element_type=jnp.float32)
         + ff + b1_ref[...])
    h = h * jax.nn.sigmoid(h)
    mix = jnp.dot(h, w2_ref[...], preferred_element_type=jnp.float32)
    mix = mix + b2_ref[...]

    fn = (messages * mix).reshape(BLK, K, D).sum(axis=1) * (1.0 / K)

    xo = f + fn
    mu = jnp.mean(xo, axis=-1, keepdims=True)
    xc = xo - mu
    var = jnp.mean(xc * xc, axis=-1, keepdims=True)
    out_ref[...] = gamma_ref[...] * xc * lax.rsqrt(var + 1e-5)
    ncoord_ref[...] = cown + 0.001 * jnp.dot(
        fn, wmove_ref[...], preferred_element_type=jnp.float32)


_OFFS = [t - 7 if t <= 6 else t - 6 for t in range(K - 1)]


def _msg_body(coord_ref, cpad_ref, feat_ref, spatf_ref, spatc_ref,
              gfe_ref, gce_ref, wtp_ref, wang_ref, bmsg_ref, w1m_ref,
              w1r_ref, w1f_ref, b1_ref, w2_ref, b2_ref, wmove_ref, gamma_ref,
              out_ref, ncoord_ref, nf_s, vc_s):
    i = pl.program_id(0)
    r0 = i * BLK
    is_edge = jnp.logical_or(i == 0, i == N // BLK - 1)

    @pl.when(jnp.logical_not(is_edge))
    def _interior():
        for tslot, doff in enumerate(_OFFS):
            nf_s[:, tslot, :] = feat_ref[pl.ds(r0 + doff, BLK), :]
            vc_s[:, tslot, :] = cpad_ref[pl.ds(r0 + doff, BLK), :]
        nf_s[:, K - 1, :] = spatf_ref[...]
        vc_s[:, K - 1, :] = spatc_ref[...][:, 0:CW]

    @pl.when(is_edge)
    def _edge():
        nf_s[...] = gfe_ref[...].reshape(BLK, K, D)
        vc_s[...] = gce_ref[...][:, 0:CW].reshape(BLK, K, CW)

    f = feat_ref[pl.ds(r0, BLK), :]
    own = cpad_ref[pl.ds(r0, BLK), :]
    nf = nf_s[...].reshape(BLK * K, D)
    vec = (vc_s[...] - own[:, None, :]).reshape(BLK * K, CW)
    _msg_tail(f, nf, vec, coord_ref[...],
              wtp_ref, wang_ref, bmsg_ref, w1m_ref, w1r_ref, w1f_ref,
              b1_ref, w2_ref, b2_ref, wmove_ref, gamma_ref,
              out_ref, ncoord_ref)


def _msg_pass(coord, cpad, features, gf, gc,
              W_tp, W_ang_p, b_msg, W1m, W1r, W1f, b1, W2, b2, W_move, gamma):
    full = lambda *shape: pl.BlockSpec(shape, lambda i: (0,) * len(shape))
    nb = N // BLK
    edge_map = lambda i: (jnp.where(i == nb - 1, 2, 1), 0)
    w_specs = [
        full(SH, D, D),
        full(SH, D),
        full(1, D),
        full(D, RB),
        full(RB, RB),
        full(D, RB),
        full(1, RB),
        full(RB, D),
        full(1, D),
        full(D, 3),
        full(1, D),
    ]
    w_args = (W_tp, W_ang_p, b_msg.reshape(1, D), W1m, W1r, W1f,
              b1.reshape(1, RB), W2, b2.reshape(1, D), W_move,
              gamma.reshape(1, D))
    return pl.pallas_call(
        _msg_body,
        grid=(nb,),
        in_specs=[
            pl.BlockSpec((BLK, 3), lambda i: (i, 0)),
            full(N, CW),
            full(N, D),
            pl.BlockSpec((BLK, D), lambda i: (i, 0)),
            pl.BlockSpec((BLK, CPAD), lambda i: (i, 0)),
            pl.BlockSpec((BLK * K, D), edge_map),
            pl.BlockSpec((BLK * K, CPAD), edge_map),
        ] + w_specs,
        out_specs=[
            pl.BlockSpec((BLK, D), lambda i: (i, 0)),
            pl.BlockSpec((BLK, 3), lambda i: (i, 0)),
        ],
        out_shape=[
            jax.ShapeDtypeStruct((N, D), jnp.float32),
            jax.ShapeDtypeStruct((N, 3), jnp.float32),
        ],
        scratch_shapes=[
            pltpu.VMEM((BLK, K, D), jnp.float32),
            pltpu.VMEM((BLK, K, CW), jnp.float32),
        ],
    )(coord, cpad, features, gf, gc, gf, gc, *w_args)


def kernel(coord, mask, features, W_tp, W_ang, b_msg, W1, b1, W2, b2,
           W_move, gamma):
    del mask
    nei = _knn_indices(coord)
    cpad = jnp.pad(coord, ((0, 0), (0, CW - 3)))
    cpad_tab = jnp.pad(coord, ((0, 0), (0, CPAD - 3)))
    idx_all = jnp.concatenate([
        nei[:, K - 1],
        nei[:BLK].reshape(BLK * K),
        nei[N - BLK:].reshape(BLK * K),
    ])
    gf, gc = _gather_neighbors(features, cpad_tab, idx_all)
    W1m, W1r, W1f = W1[:D], W1[D:D + RB], W1[D + RB:]
    out, new_coord = _msg_pass(
        coord, cpad, features, gf, gc,
        W_tp.astype(jnp.bfloat16), W_ang, b_msg, W1m, W1r, W1f, b1, W2, b2,
        W_move, gamma)
    return out, new_coord
